# Initial kernel scaffold; baseline (speedup 1.0000x reference)
import jax
import jax.numpy as jnp
from jax.experimental import pallas as pl


def kernel(x, emb_table):
    raise NotImplementedError("write your pallas kernel here")


# speedup vs baseline: 2.8565x; 2.8565x over previous
import jax
import jax.numpy as jnp
from jax.experimental import pallas as pl


_BS = 512


def _add_kernel(x_ref, emb_ref, out_ref):
    out_ref[...] = x_ref[...] + emb_ref[...][None, :, :]


def kernel(x, emb_table):
    B, S, D = x.shape
    num_s = S // _BS
    return pl.pallas_call(
        _add_kernel,
        grid=(num_s, B),
        in_specs=[
            pl.BlockSpec((1, _BS, D), lambda i, j: (j, i, 0)),
            pl.BlockSpec((_BS, D), lambda i, j: (i, 0)),
        ],
        out_specs=pl.BlockSpec((1, _BS, D), lambda i, j: (j, i, 0)),
        out_shape=jax.ShapeDtypeStruct((B, S, D), x.dtype),
    )(x, emb_table)


# speedup vs baseline: 3.1816x; 1.1138x over previous
import jax
import jax.numpy as jnp
from jax.experimental import pallas as pl


_BS = 1024


def _add_kernel(x_ref, emb_ref, out_ref):
    out_ref[...] = x_ref[...] + emb_ref[...][None, :, :]


def kernel(x, emb_table):
    B, S, D = x.shape
    num_s = S // _BS
    return pl.pallas_call(
        _add_kernel,
        grid=(num_s, B),
        in_specs=[
            pl.BlockSpec((1, _BS, D), lambda i, j: (j, i, 0)),
            pl.BlockSpec((_BS, D), lambda i, j: (i, 0)),
        ],
        out_specs=pl.BlockSpec((1, _BS, D), lambda i, j: (j, i, 0)),
        out_shape=jax.ShapeDtypeStruct((B, S, D), x.dtype),
    )(x, emb_table)


# speedup vs baseline: 3.3072x; 1.0395x over previous
import jax
import jax.numpy as jnp
from jax.experimental import pallas as pl


_BS = 2048


def _add_kernel(x_ref, emb_ref, out_ref):
    out_ref[...] = x_ref[...] + emb_ref[...][None, :, :]


def kernel(x, emb_table):
    B, S, D = x.shape
    num_s = S // _BS
    return pl.pallas_call(
        _add_kernel,
        grid=(num_s, B),
        in_specs=[
            pl.BlockSpec((1, _BS, D), lambda i, j: (j, i, 0)),
            pl.BlockSpec((_BS, D), lambda i, j: (i, 0)),
        ],
        out_specs=pl.BlockSpec((1, _BS, D), lambda i, j: (j, i, 0)),
        out_shape=jax.ShapeDtypeStruct((B, S, D), x.dtype),
    )(x, emb_table)
